# Initial kernel scaffold; baseline (speedup 1.0000x reference)
import jax
import jax.numpy as jnp
from jax.experimental import pallas as pl


def kernel(numerical_data, cat_data, tables, W1, b1, W2, b2, Wo, bo, g0, be0, g1, be1, g2, be2):
    raise NotImplementedError("write your pallas kernel here")


# speedup vs baseline: 7.4874x; 7.4874x over previous
import functools

import jax
import jax.numpy as jnp
from jax import lax
from jax.experimental import pallas as pl
from jax.experimental.pallas import tpu as pltpu
from jax.experimental.pallas import tpu_sc as plsc

EPS = 1e-5
_NW = 32
_LANES = 128


def _sc_gather(table, idx, C):
    nidx = idx.shape[0]
    Dd = table.shape[1]
    per_w = nidx // _NW
    n_chunk = per_w // C
    mesh = plsc.VectorSubcoreMesh(core_axis_name="c", subcore_axis_name="s")

    @functools.partial(
        pl.kernel,
        mesh=mesh,
        compiler_params=pltpu.CompilerParams(use_tc_tiling_on_sc=False),
        out_type=jax.ShapeDtypeStruct((nidx, Dd), jnp.float32),
        scratch_types=[
            pltpu.VMEM((per_w,), jnp.int32),
            pltpu.VMEM((C, Dd), jnp.float32),
            pltpu.SemaphoreType.DMA,
        ],
    )
    def gather_k(table_hbm, idx_hbm, out_hbm, idx_v, rows_v, gsem):
        wid = lax.axis_index("s") * 2 + lax.axis_index("c")
        base = wid * per_w
        pltpu.sync_copy(idx_hbm.at[pl.ds(base, per_w)], idx_v)

        def body(j, carry):
            pltpu.async_copy(
                table_hbm.at[idx_v.at[pl.ds(j * C, C)]], rows_v, gsem
            ).wait()
            pltpu.sync_copy(rows_v, out_hbm.at[pl.ds(base + j * C, C)])
            return carry

        lax.fori_loop(0, n_chunk, body, 0)

    return gather_k(table, idx)


def _num_bn_body(num_ref, g_ref, b_ref, out_ref):
    x = num_ref[...]
    mean = jnp.mean(x, axis=0, keepdims=True)
    var = jnp.mean((x - mean) ** 2, axis=0, keepdims=True)
    out_ref[...] = (x - mean) * lax.rsqrt(var + EPS) * g_ref[...] + b_ref[...]


def _l1_body(emb_ref, num_ref, w1e_ref, w1n_ref, b1_ref, h1_ref, acc_ref):
    i = pl.program_id(0)
    h = jnp.dot(emb_ref[...], w1e_ref[...], preferred_element_type=jnp.float32)
    h = h + jnp.dot(num_ref[...], w1n_ref[...], preferred_element_type=jnp.float32)
    h = jnp.maximum(h + b1_ref[...], 0.0)
    h1_ref[...] = h
    stats = jnp.concatenate(
        [jnp.sum(h, axis=0, keepdims=True), jnp.sum(h * h, axis=0, keepdims=True)],
        axis=0,
    )

    @pl.when(i == 0)
    def _():
        acc_ref[...] = stats

    @pl.when(i > 0)
    def _():
        acc_ref[...] += stats


def _l2_body(h1_ref, st_ref, g_ref, be_ref, w2_ref, b2_ref, h2_ref, acc_ref, *, nB):
    i = pl.program_id(0)
    mean = st_ref[0:1, :] * (1.0 / nB)
    var = st_ref[1:2, :] * (1.0 / nB) - mean * mean
    xn = (h1_ref[...] - mean) * lax.rsqrt(var + EPS) * g_ref[...] + be_ref[...]
    h = jnp.dot(xn, w2_ref[...], preferred_element_type=jnp.float32)
    h = jnp.maximum(h + b2_ref[...], 0.0)
    h2_ref[...] = h
    stats = jnp.concatenate(
        [jnp.sum(h, axis=0, keepdims=True), jnp.sum(h * h, axis=0, keepdims=True)],
        axis=0,
    )

    @pl.when(i == 0)
    def _():
        acc_ref[...] = stats

    @pl.when(i > 0)
    def _():
        acc_ref[...] += stats


def _l3_body(h2_ref, st_ref, g_ref, be_ref, wo_ref, bo_ref, out_ref, *, nB):
    mean = st_ref[0:1, :] * (1.0 / nB)
    var = st_ref[1:2, :] * (1.0 / nB) - mean * mean
    xn = (h2_ref[...] - mean) * lax.rsqrt(var + EPS) * g_ref[...] + be_ref[...]
    out_ref[...] = jnp.sum(xn * wo_ref[...], axis=1, keepdims=True) + bo_ref[...]


def kernel(numerical_data, cat_data, tables, W1, b1, W2, b2, Wo, bo,
           g0, be0, g1, be1, g2, be2):
    B, NUM = numerical_data.shape
    F = cat_data.shape[1]
    V = tables.shape[1]
    D = tables.shape[2]
    ED = F * D
    H1, H2 = W1.shape[0], W2.shape[0]
    fB = float(B)

    table_flat = tables.reshape(F * V, D)
    flat_idx = (
        cat_data.astype(jnp.int32) + (jnp.arange(F, dtype=jnp.int32) * V)[None, :]
    ).reshape(B * F)
    emb = _sc_gather(table_flat, flat_idx, C=1024).reshape(B, ED)

    numn = pl.pallas_call(
        _num_bn_body,
        out_shape=jax.ShapeDtypeStruct((B, NUM), jnp.float32),
    )(numerical_data, g0.reshape(1, NUM), be0.reshape(1, NUM))

    bt = 1024
    T = B // bt

    h1, st1 = pl.pallas_call(
        _l1_body,
        grid=(T,),
        in_specs=[
            pl.BlockSpec((bt, ED), lambda i: (i, 0)),
            pl.BlockSpec((bt, NUM), lambda i: (i, 0)),
            pl.BlockSpec((ED, H1), lambda i: (0, 0)),
            pl.BlockSpec((NUM, H1), lambda i: (0, 0)),
            pl.BlockSpec((1, H1), lambda i: (0, 0)),
        ],
        out_specs=[
            pl.BlockSpec((bt, H1), lambda i: (i, 0)),
            pl.BlockSpec((2, H1), lambda i: (0, 0)),
        ],
        out_shape=[
            jax.ShapeDtypeStruct((B, H1), jnp.float32),
            jax.ShapeDtypeStruct((2, H1), jnp.float32),
        ],
    )(emb, numn, W1[:, :ED].T, W1[:, ED:].T, b1.reshape(1, H1))

    h2, st2 = pl.pallas_call(
        functools.partial(_l2_body, nB=fB),
        grid=(T,),
        in_specs=[
            pl.BlockSpec((bt, H1), lambda i: (i, 0)),
            pl.BlockSpec((2, H1), lambda i: (0, 0)),
            pl.BlockSpec((1, H1), lambda i: (0, 0)),
            pl.BlockSpec((1, H1), lambda i: (0, 0)),
            pl.BlockSpec((H1, H2), lambda i: (0, 0)),
            pl.BlockSpec((1, H2), lambda i: (0, 0)),
        ],
        out_specs=[
            pl.BlockSpec((bt, H2), lambda i: (i, 0)),
            pl.BlockSpec((2, H2), lambda i: (0, 0)),
        ],
        out_shape=[
            jax.ShapeDtypeStruct((B, H2), jnp.float32),
            jax.ShapeDtypeStruct((2, H2), jnp.float32),
        ],
    )(h1, st1, g1.reshape(1, H1), be1.reshape(1, H1), W2.T, b2.reshape(1, H2))

    out = pl.pallas_call(
        functools.partial(_l3_body, nB=fB),
        grid=(T,),
        in_specs=[
            pl.BlockSpec((bt, H2), lambda i: (i, 0)),
            pl.BlockSpec((2, H2), lambda i: (0, 0)),
            pl.BlockSpec((1, H2), lambda i: (0, 0)),
            pl.BlockSpec((1, H2), lambda i: (0, 0)),
            pl.BlockSpec((1, H2), lambda i: (0, 0)),
            pl.BlockSpec((1, 1), lambda i: (0, 0)),
        ],
        out_specs=pl.BlockSpec((bt, 1), lambda i: (i, 0)),
        out_shape=jax.ShapeDtypeStruct((B, 1), jnp.float32),
    )(h2, st2, g2.reshape(1, H2), be2.reshape(1, H2), Wo.reshape(1, H2),
      bo.reshape(1, 1))

    return out
